# baseline (speedup 1.0000x reference)
import functools

import jax
import jax.numpy as jnp
from jax import lax
from jax.experimental import pallas as pl
from jax.experimental.pallas import tpu as pltpu
from jax.experimental.pallas import tpu_sc as plsc

DIM = 32
NC = 2
NS = 16
L = 16
NW = NC * NS
C = 128


@functools.partial(jax.jit, static_argnames=("tot",))
def _sc_scores(h, r, t, ent_re, ent_im, rel_re, rel_im, tot):
    per_w = tot // NW
    n_chunks = per_w // C
    mesh = plsc.VectorSubcoreMesh(core_axis_name="c", subcore_axis_name="s")

    @functools.partial(
        pl.kernel,
        out_type=jax.ShapeDtypeStruct((tot,), jnp.float32),
        mesh=mesh,
        compiler_params=pltpu.CompilerParams(
            needs_layout_passes=False, use_tc_tiling_on_sc=False),
        scratch_types=[
            pltpu.VMEM((C,), jnp.int32),
            pltpu.VMEM((C,), jnp.int32),
            pltpu.VMEM((C,), jnp.int32),
            pltpu.VMEM((C, DIM // 2), jnp.int32),
            pltpu.VMEM((C, DIM // 2), jnp.int32),
            pltpu.VMEM((C, DIM // 2), jnp.int32),
            pltpu.VMEM((C, DIM // 2), jnp.int32),
            pltpu.VMEM((C, DIM // 2), jnp.int32),
            pltpu.VMEM((C, DIM // 2), jnp.int32),
            pltpu.VMEM((C,), jnp.float32),
            pltpu.SemaphoreType.DMA,
        ],
    )
    def k(h_hbm, r_hbm, t_hbm, ere_hbm, eim_hbm, rre_hbm, rim_hbm, out_hbm,
          h_v, r_v, t_v, hre, him, tre, tim, rre, rim, sc_v, sem):
        wid = lax.axis_index("s") * NC + lax.axis_index("c")
        w_base = wid * per_w
        ereb, eimb, rreb, rimb = ere_hbm, eim_hbm, rre_hbm, rim_hbm

        def chunk_body(ci, carry):
            base = w_base + ci * C
            pltpu.sync_copy(h_hbm.at[pl.ds(base, C)], h_v)
            pltpu.sync_copy(r_hbm.at[pl.ds(base, C)], r_v)
            pltpu.sync_copy(t_hbm.at[pl.ds(base, C)], t_v)
            cps = [
                pltpu.async_copy(ereb.at[h_v], hre, sem),
                pltpu.async_copy(eimb.at[h_v], him, sem),
                pltpu.async_copy(ereb.at[t_v], tre, sem),
                pltpu.async_copy(eimb.at[t_v], tim, sem),
                pltpu.async_copy(rreb.at[r_v], rre, sem),
                pltpu.async_copy(rimb.at[r_v], rim, sem),
            ]
            for cp in cps:
                cp.wait()

            def unpk(ref, row, col):
                w = plsc.load_gather(ref, [row, col])
                return plsc.unpack(plsc.bitcast(w, jnp.bfloat16),
                                   format=plsc.PackFormat.INTERLEAVED)

            def group_body(g, gcarry):
                row = g * L + lax.iota(jnp.int32, L)
                acc = jnp.zeros((L,), jnp.float32)
                for d2 in range(DIM // 2):
                    col = jnp.full((L,), d2, jnp.int32)
                    a0, a1 = unpk(hre, row, col)
                    b0, b1 = unpk(him, row, col)
                    u0, u1 = unpk(tre, row, col)
                    v0, v1 = unpk(tim, row, col)
                    p0, p1 = unpk(rre, row, col)
                    q0, q1 = unpk(rim, row, col)
                    acc = acc + p0 * (a0 * u0 + b0 * v0) + q0 * (a0 * v0 - b0 * u0)
                    acc = acc + p1 * (a1 * u1 + b1 * v1) + q1 * (a1 * v1 - b1 * u1)
                sc_v[pl.ds(g * L, L)] = acc
                return gcarry

            lax.fori_loop(0, C // L, group_body, 0)
            pltpu.sync_copy(sc_v, out_hbm.at[pl.ds(base, C)])
            return carry

        lax.fori_loop(0, n_chunks, chunk_body, 0)

    return k(h, r, t, ent_re, ent_im, rel_re, rel_im)


def kernel(pos_triples, neg_triples, ent_re, ent_im, rel_re, rel_im):
    tri = jnp.concatenate([pos_triples, neg_triples], axis=0).astype(jnp.int32)
    tot = tri.shape[0]
    def _words(x):
        return jax.lax.bitcast_convert_type(
            x.astype(jnp.bfloat16).reshape(-1, DIM // 2, 2), jnp.int32)

    out = _sc_scores(tri[:, 0], tri[:, 1], tri[:, 2],
                     _words(ent_re), _words(ent_im),
                     _words(rel_re), _words(rel_im), tot)
    b = pos_triples.shape[0]
    return out[:b], out[b:]


# speedup vs baseline: 2.2230x; 2.2230x over previous
import functools

import jax
import jax.numpy as jnp
from jax import lax
from jax.experimental import pallas as pl
from jax.experimental.pallas import tpu as pltpu
from jax.experimental.pallas import tpu_sc as plsc

DIM = 32
NC = 2
NS = 16
L = 16
NW = NC * NS
C = 128


@functools.partial(jax.jit, static_argnames=("tot",))
def _sc_scores(h, r, t, ent_re, ent_im, rel_re, rel_im, tot):
    per_w = tot // NW
    n_chunks = per_w // C
    mesh = plsc.VectorSubcoreMesh(core_axis_name="c", subcore_axis_name="s")

    @functools.partial(
        pl.kernel,
        out_type=jax.ShapeDtypeStruct((tot,), jnp.float32),
        mesh=mesh,
        compiler_params=pltpu.CompilerParams(
            needs_layout_passes=False, use_tc_tiling_on_sc=False),
        scratch_types=[
            pltpu.VMEM((C,), jnp.int32),
            pltpu.VMEM((C,), jnp.int32),
            pltpu.VMEM((C,), jnp.int32),
            pltpu.VMEM((C, DIM), jnp.float32),
            pltpu.VMEM((C, DIM), jnp.float32),
            pltpu.VMEM((C, DIM), jnp.float32),
            pltpu.VMEM((C, DIM), jnp.float32),
            pltpu.VMEM((C, DIM), jnp.float32),
            pltpu.VMEM((C, DIM), jnp.float32),
            pltpu.VMEM((C,), jnp.float32),
            pltpu.SemaphoreType.DMA,
        ],
    )
    def k(h_hbm, r_hbm, t_hbm, ere_hbm, eim_hbm, rre_hbm, rim_hbm, out_hbm,
          h_v, r_v, t_v, hre, him, tre, tim, rre, rim, sc_v, sem):
        wid = lax.axis_index("s") * NC + lax.axis_index("c")
        w_base = wid * per_w

        def chunk_body(ci, carry):
            base = w_base + ci * C
            pltpu.sync_copy(h_hbm.at[pl.ds(base, C)], h_v)
            pltpu.sync_copy(r_hbm.at[pl.ds(base, C)], r_v)
            pltpu.sync_copy(t_hbm.at[pl.ds(base, C)], t_v)
            cps = [
                pltpu.async_copy(ere_hbm.at[h_v], hre, sem),
                pltpu.async_copy(eim_hbm.at[h_v], him, sem),
                pltpu.async_copy(ere_hbm.at[t_v], tre, sem),
                pltpu.async_copy(eim_hbm.at[t_v], tim, sem),
                pltpu.async_copy(rre_hbm.at[r_v], rre, sem),
                pltpu.async_copy(rim_hbm.at[r_v], rim, sem),
            ]
            for cp in cps:
                cp.wait()

            def group_body(g, gcarry):
                row = g * L + lax.iota(jnp.int32, L)
                acc = jnp.zeros((L,), jnp.float32)
                for d in range(DIM):
                    col = jnp.full((L,), d, jnp.int32)
                    a = plsc.load_gather(hre, [row, col])
                    b = plsc.load_gather(him, [row, col])
                    u = plsc.load_gather(tre, [row, col])
                    v = plsc.load_gather(tim, [row, col])
                    p = plsc.load_gather(rre, [row, col])
                    q = plsc.load_gather(rim, [row, col])
                    acc = acc + p * (a * u + b * v) + q * (a * v - b * u)
                sc_v[pl.ds(g * L, L)] = acc
                return gcarry

            lax.fori_loop(0, C // L, group_body, 0)
            pltpu.sync_copy(sc_v, out_hbm.at[pl.ds(base, C)])
            return carry

        lax.fori_loop(0, n_chunks, chunk_body, 0)

    return k(h, r, t, ent_re, ent_im, rel_re, rel_im)


def kernel(pos_triples, neg_triples, ent_re, ent_im, rel_re, rel_im):
    tri = jnp.concatenate([pos_triples, neg_triples], axis=0).astype(jnp.int32)
    tot = tri.shape[0]
    out = _sc_scores(tri[:, 0], tri[:, 1], tri[:, 2],
                     ent_re, ent_im, rel_re, rel_im, tot)
    b = pos_triples.shape[0]
    return out[:b], out[b:]
